# baseline (speedup 1.0000x reference)
import jax
import jax.numpy as jnp
from jax.experimental import pallas as pl
from jax.experimental.pallas import tpu as pltpu

_LANES = 128
_BS = 8


def _body(t_ref, alpha_ref, x_ref, n_ref, out_ref):
    b0 = pl.program_id(0) * _BS
    for r in range(_BS):
        a = alpha_ref[t_ref[b0 + r]]
        sa = jnp.sqrt(a)
        sn = jnp.sqrt(1.0 - a)
        out_ref[r] = sa * x_ref[r] + sn * n_ref[r]


def kernel(x_0, noise, t, alpha_cumprod):
    B, C, H, W = x_0.shape
    out = pl.pallas_call(
        _body,
        grid=(B // _BS,),
        in_specs=[
            pl.BlockSpec(memory_space=pltpu.SMEM),
            pl.BlockSpec(memory_space=pltpu.SMEM),
            pl.BlockSpec((_BS, C, H, W), lambda b: (b, 0, 0, 0)),
            pl.BlockSpec((_BS, C, H, W), lambda b: (b, 0, 0, 0)),
        ],
        out_specs=pl.BlockSpec((_BS, C, H, W), lambda b: (b, 0, 0, 0)),
        out_shape=jax.ShapeDtypeStruct((B, C, H, W), x_0.dtype),
    )(t, alpha_cumprod, x_0, noise)
    return (out, noise, t)


# speedup vs baseline: 3.7132x; 3.7132x over previous
import jax
import jax.numpy as jnp
from jax.experimental import pallas as pl
from jax.experimental.pallas import tpu as pltpu

_HB = 16


def _body(a_ref, x_ref, n_ref, out_ref, nout_ref):
    a = a_ref[...].reshape(1, 1, 1, 128)
    sa = jnp.sqrt(a)
    sn = jnp.sqrt(1.0 - a)
    n = n_ref[...]
    out_ref[...] = sa * x_ref[...] + sn * n
    nout_ref[...] = n


def kernel(x_0, noise, t, alpha_cumprod):
    B, C, H, W = x_0.shape
    xT = jnp.transpose(x_0, (1, 2, 3, 0))
    nT = jnp.transpose(noise, (1, 2, 3, 0))
    a_vec = jnp.take(alpha_cumprod, t, axis=0).reshape(1, B)
    blk = (C, _HB, W, B)
    bmap = lambda h: (0, h, 0, 0)
    outT, noutT = pl.pallas_call(
        _body,
        grid=(H // _HB,),
        in_specs=[
            pl.BlockSpec((1, B), lambda h: (0, 0)),
            pl.BlockSpec(blk, bmap),
            pl.BlockSpec(blk, bmap),
        ],
        out_specs=[pl.BlockSpec(blk, bmap), pl.BlockSpec(blk, bmap)],
        out_shape=[
            jax.ShapeDtypeStruct((C, H, W, B), x_0.dtype),
            jax.ShapeDtypeStruct((C, H, W, B), x_0.dtype),
        ],
    )(a_vec, xT, nT)
    return (
        jnp.transpose(outT, (3, 0, 1, 2)),
        jnp.transpose(noutT, (3, 0, 1, 2)),
        t,
    )
